# baseline (speedup 1.0000x reference)
import functools

import jax
import jax.numpy as jnp
from jax import lax
from jax.experimental import pallas as pl
from jax.experimental.pallas import tpu as pltpu
from jax.experimental.pallas import tpu_sc as plsc

VOCAB = 8192
D = 8192
N = 8192
NC, NS = 2, 16
NW = NC * NS
TPW = N // NW
R = 2
STEPS = TPW // R
NBUF = 4
LAG = 2


def _body(table_hbm, idx_hbm, out_hbm, idx_v, bufs, gsems, ssems):
    wid = lax.axis_index("s") * NC + lax.axis_index("c")
    base = wid * TPW

    pltpu.sync_copy(idx_hbm.at[wid], idx_v)

    def start_gather(s, b):
        pltpu.make_async_copy(table_hbm.at[idx_v.at[s]], bufs[b], gsems[b]).start()

    def wait_gather(b):
        pltpu.make_async_copy(table_hbm.at[idx_v.at[0]], bufs[b], gsems[b]).wait()

    def start_scatter(s, b):
        pltpu.make_async_copy(
            bufs[b], out_hbm.at[pl.ds(base + s * R, R)], ssems[b]
        ).start()

    def wait_scatter(b):
        pltpu.make_async_copy(
            bufs[b], out_hbm.at[pl.ds(base, R)], ssems[b]
        ).wait()

    for b in range(NBUF):
        start_gather(b, b)

    def step(s_base, b):
        s = s_base + b
        wait_gather(b)
        start_scatter(s, b)
        bq = (b - LAG) % NBUF
        sq = s - LAG
        sn = sq + NBUF

        @pl.when(jnp.logical_and(sq >= 0, sn < STEPS))
        def _():
            wait_scatter(bq)
            start_gather(sn, bq)

    def outer(r, carry):
        for b in range(NBUF):
            step(r * NBUF, b)
        return carry

    lax.fori_loop(0, STEPS // NBUF, outer, 0)

    for b in range(NBUF):
        s_last = STEPS - NBUF + b
        bq = s_last % NBUF
        wait_scatter(bq)


@functools.partial(jax.jit, static_argnames=())
def kernel(idx, embedding):
    B, L = idx.shape
    idx3 = idx.reshape(NW, STEPS, R).astype(jnp.int32)

    mesh = plsc.VectorSubcoreMesh(
        core_axis_name="c", subcore_axis_name="s", num_cores=NC, num_subcores=NS
    )

    def body(table_hbm, idx_hbm, out_hbm, *scratch):
        bufs = scratch[1 : 1 + NBUF]
        gsems = scratch[1 + NBUF : 1 + 2 * NBUF]
        ssems = scratch[1 + 2 * NBUF :]
        _body(table_hbm, idx_hbm, out_hbm, scratch[0], bufs, gsems, ssems)

    out = pl.kernel(
        body,
        out_type=jax.ShapeDtypeStruct((N, D), jnp.float32),
        mesh=mesh,
        scratch_types=(
            [pltpu.VMEM((STEPS, R), jnp.int32)]
            + [pltpu.VMEM((R, D), jnp.float32) for _ in range(NBUF)]
            + [pltpu.SemaphoreType.DMA for _ in range(2 * NBUF)]
        ),
    )(embedding, idx3)
    return out.reshape(B, L, D)


# speedup vs baseline: 1.0070x; 1.0070x over previous
import functools

import jax
import jax.numpy as jnp
from jax import lax
from jax.experimental import pallas as pl
from jax.experimental.pallas import tpu as pltpu
from jax.experimental.pallas import tpu_sc as plsc

VOCAB = 8192
D = 8192
N = 8192
NC, NS = 2, 16
NW = NC * NS
TPW = N // NW
R = 4
STEPS = TPW // R
NBUF = 3
LAG = 1


def _body(table_hbm, idx_hbm, out_hbm, idx_v, bufs, gsems, ssems):
    wid = lax.axis_index("s") * NC + lax.axis_index("c")
    base = wid * TPW

    pltpu.sync_copy(idx_hbm.at[wid], idx_v)

    def start_gather(s, b):
        pltpu.make_async_copy(table_hbm.at[idx_v.at[s]], bufs[b], gsems[b]).start()

    def wait_gather(b):
        pltpu.make_async_copy(table_hbm.at[idx_v.at[0]], bufs[b], gsems[b]).wait()

    def start_scatter(s, b):
        pltpu.make_async_copy(
            bufs[b], out_hbm.at[pl.ds(base + s * R, R)], ssems[b]
        ).start()

    def wait_scatter(b):
        pltpu.make_async_copy(
            bufs[b], out_hbm.at[pl.ds(base, R)], ssems[b]
        ).wait()

    for b in range(NBUF):
        start_gather(b, b)

    def step(s, b, static):
        wait_gather(b)
        start_scatter(s, b)
        bq = (b - LAG) % NBUF
        sq = s - LAG
        sn = sq + NBUF

        def retire():
            wait_scatter(bq)
            start_gather(sn, bq)

        if static:
            if sq >= 0 and sn < STEPS:
                retire()
        else:
            pl.when(jnp.logical_and(sq >= 0, sn < STEPS))(retire)

    G = STEPS // NBUF

    def outer(r, carry):
        for b in range(NBUF):
            step(r * NBUF + b, b, static=False)
        return carry

    lax.fori_loop(0, G, outer, 0)

    for s in range(G * NBUF, STEPS):
        step(s, s % NBUF, static=True)

    for s in range(STEPS - NBUF, STEPS):
        wait_scatter(s % NBUF)


@functools.partial(jax.jit, static_argnames=())
def kernel(idx, embedding):
    B, L = idx.shape
    idx3 = idx.reshape(NW, STEPS, R).astype(jnp.int32)

    mesh = plsc.VectorSubcoreMesh(
        core_axis_name="c", subcore_axis_name="s", num_cores=NC, num_subcores=NS
    )

    def body(table_hbm, idx_hbm, out_hbm, *scratch):
        bufs = scratch[1 : 1 + NBUF]
        gsems = scratch[1 + NBUF : 1 + 2 * NBUF]
        ssems = scratch[1 + 2 * NBUF :]
        _body(table_hbm, idx_hbm, out_hbm, scratch[0], bufs, gsems, ssems)

    out = pl.kernel(
        body,
        out_type=jax.ShapeDtypeStruct((N, D), jnp.float32),
        mesh=mesh,
        scratch_types=(
            [pltpu.VMEM((STEPS, R), jnp.int32)]
            + [pltpu.VMEM((R, D), jnp.float32) for _ in range(NBUF)]
            + [pltpu.SemaphoreType.DMA for _ in range(2 * NBUF)]
        ),
    )(embedding, idx3)
    return out.reshape(B, L, D)
